# baseline (speedup 1.0000x reference)
import jax
import jax.numpy as jnp
from jax import lax
from jax.experimental import pallas as pl
from jax.experimental.pallas import tpu as pltpu
from jax.experimental.pallas import tpu_sc as plsc

_CLASS_IDS = (3, 4, 5, 6, 7, 11, 16, 25, 32, 35)
_K = len(_CLASS_IDS)
_KH = _K // 2
_H = _W = 1024
_NC, _NS, _L = 2, 16, 16
_NW = _NC * _NS
_ROWS_W = _H // _NW
_R = 8
_SLABS = _ROWS_W // _R
_CG = _W // _L


def _onehot_body(lab_hbm, out_hbm, *refs):
    labs = refs[0:2]
    sets = (refs[2:2 + _KH], refs[2 + _KH:2 + 2 * _KH])
    in_sems = refs[2 + 2 * _KH:4 + 2 * _KH]
    out_sems = refs[4 + 2 * _KH:6 + 2 * _KH]

    wid = lax.axis_index("s") * _NC + lax.axis_index("c")
    row0 = wid * _ROWS_W
    ones = jnp.full((_L,), 1.0, jnp.float32)
    zeros = jnp.zeros((_L,), jnp.float32)

    def fetch(s):
        return pltpu.async_copy(
            lab_hbm.at[pl.ds(row0 + s * _R, _R), :], labs[s % 2], in_sems[s % 2])

    in_descs = {0: fetch(0)}
    out_descs = {}

    item = 0
    for s in range(_SLABS):
        if s + 1 < _SLABS:
            in_descs[s + 1] = fetch(s + 1)
        in_descs[s].wait()
        lab_v = labs[s % 2]

        for half in range(2):
            cids = _CLASS_IDS[half * _KH:(half + 1) * _KH]
            bufs = sets[item % 2]
            if item >= 2:
                for d in out_descs[item - 2]:
                    d.wait()

            @plsc.parallel_loop(0, _CG, unroll=4)
            def g_body(cg):
                c = cg * _L
                for r in range(_R):
                    v = lab_v[r, pl.ds(c, _L)]
                    for j, cid in enumerate(cids):
                        bufs[j][r, pl.ds(c, _L)] = jnp.where(v == cid, ones, zeros)

            out_descs[item] = [
                pltpu.async_copy(
                    bufs[j],
                    out_hbm.at[half * _KH + j, pl.ds(row0 + s * _R, _R), :],
                    out_sems[item % 2],
                )
                for j in range(_KH)
            ]
            item += 1

    for i in (item - 2, item - 1):
        for d in out_descs[i]:
            d.wait()


def kernel(label):
    lab = label.astype(jnp.int32)
    return pl.kernel(
        _onehot_body,
        out_type=jax.ShapeDtypeStruct((_K, _H, _W), jnp.float32),
        mesh=plsc.VectorSubcoreMesh(
            core_axis_name="c", subcore_axis_name="s",
            num_cores=_NC, num_subcores=_NS,
        ),
        scratch_types=(
            [pltpu.VMEM((_R, _W), jnp.int32)] * 2
            + [pltpu.VMEM((_R, _W), jnp.float32)] * (2 * _KH)
            + [pltpu.SemaphoreType.DMA] * 4
        ),
    )(lab)


# speedup vs baseline: 1.3344x; 1.3344x over previous
import jax
import jax.numpy as jnp
from jax import lax
from jax.experimental import pallas as pl
from jax.experimental.pallas import tpu as pltpu
from jax.experimental.pallas import tpu_sc as plsc

_CLASS_IDS = (3, 4, 5, 6, 7, 11, 16, 25, 32, 35)
_K = len(_CLASS_IDS)
_H = _W = 1024
_NC, _NS, _L = 2, 16, 16
_NW = _NC * _NS
_ROWS_W = _H // _NW
_R = 4
_CHUNKS = _ROWS_W // _R
_GROUPS = _R * _W // _L


def _onehot_body(lab_hbm, out_hbm,
                 lab0, lab1, out0, out1,
                 in_sem0, in_sem1, out_sem0, out_sem1):
    wid = lax.axis_index("s") * _NC + lax.axis_index("c")
    row0 = wid * _ROWS_W
    ones = jnp.full((_L,), 1.0, jnp.float32)
    zeros = jnp.zeros((_L,), jnp.float32)
    labs = (lab0, lab1)
    outs = (out0, out1)
    in_sems = (in_sem0, in_sem1)
    out_sems = (out_sem0, out_sem1)

    def fetch(t):
        return pltpu.async_copy(
            lab_hbm.at[pl.ds(row0 + t * _R, _R), :], labs[t % 2], in_sems[t % 2])

    in_descs = {0: fetch(0)}
    out_descs = {}

    for t in range(_CHUNKS):
        b = t % 2
        if t + 1 < _CHUNKS:
            in_descs[t + 1] = fetch(t + 1)
        in_descs[t].wait()
        if t >= 2:
            for d in out_descs[t - 2]:
                d.wait()

        lab_v, out_v = labs[b], outs[b]

        def g_body(g, carry):
            r = g // (_W // _L)
            c = (g % (_W // _L)) * _L
            v = lab_v[r, pl.ds(c, _L)]
            for k, cid in enumerate(_CLASS_IDS):
                out_v[k, r, pl.ds(c, _L)] = jnp.where(v == cid, ones, zeros)
            return carry

        lax.fori_loop(0, _GROUPS, g_body, 0)

        out_descs[t] = [
            pltpu.async_copy(
                out_v.at[k],
                out_hbm.at[k, pl.ds(row0 + t * _R, _R), :],
                out_sems[b],
            )
            for k in range(_K)
        ]

    for t in (_CHUNKS - 2, _CHUNKS - 1):
        for d in out_descs[t]:
            d.wait()


def kernel(label):
    lab = label.astype(jnp.int32)
    return pl.kernel(
        _onehot_body,
        out_type=jax.ShapeDtypeStruct((_K, _H, _W), jnp.float32),
        mesh=plsc.VectorSubcoreMesh(
            core_axis_name="c", subcore_axis_name="s",
            num_cores=_NC, num_subcores=_NS,
        ),
        scratch_types=[
            pltpu.VMEM((_R, _W), jnp.int32),
            pltpu.VMEM((_R, _W), jnp.int32),
            pltpu.VMEM((_K, _R, _W), jnp.float32),
            pltpu.VMEM((_K, _R, _W), jnp.float32),
            pltpu.SemaphoreType.DMA,
            pltpu.SemaphoreType.DMA,
            pltpu.SemaphoreType.DMA,
            pltpu.SemaphoreType.DMA,
        ],
    )(lab)
